# baseline (speedup 1.0000x reference)
import functools

import jax
import jax.numpy as jnp
from jax import lax
from jax.experimental import pallas as pl
from jax.experimental.pallas import tpu as pltpu
from jax.experimental.pallas import tpu_sc as plsc

N = 10000
E = 320000
G = 256
NV = 100
EV = 20
EVP = 32
D = 128
LATENT = 64
HOPS = 3

F32 = jnp.float32


def _dg00(a, b):
    return lax.dot_general(a, b, (((0,), (0,)), ((), ())),
                           preferred_element_type=F32)


def _dg11(a, b):
    return lax.dot_general(a, b, (((1,), (1,)), ((), ())),
                           preferred_element_type=F32)


def _tc_init_body(nt_ref, nemb_ref, eemb_ref, wmsg_ref,
                  hT_ref, hmT_ref, ebT_ref):
    nt = nt_ref[...]
    rows = lax.broadcasted_iota(jnp.int32, (NV, N), 0)
    ohT = (rows == nt).astype(F32)
    hT = _dg00(nemb_ref[...], ohT)
    wm1 = wmsg_ref[0:D, :]
    wm2 = wmsg_ref[D:2 * D, :]
    hT_ref[...] = hT
    hmT_ref[...] = _dg00(wm1, hT)
    ebT_ref[...] = lax.dot_general(wm2, eemb_ref[...],
                                   (((0,), (1,)), ((), ())),
                                   preferred_element_type=F32)


def _tc_init(nt2, node_embed, edge_embed_pad, W_msg):
    return pl.pallas_call(
        _tc_init_body,
        out_shape=[
            jax.ShapeDtypeStruct((D, N), F32),
            jax.ShapeDtypeStruct((D, N), F32),
            jax.ShapeDtypeStruct((D, EVP), F32),
        ],
    )(nt2, node_embed, edge_embed_pad, W_msg)


def _tc_update_body(h_ref, agg_ref, wupd_ref, wmsg_ref, h2_ref, hm2_ref):
    wu1 = wupd_ref[0:D, :]
    wu2 = wupd_ref[D:2 * D, :]
    h2 = jnp.maximum(_dg00(wu1, h_ref[...]) + _dg00(wu2, agg_ref[...]), 0.0)
    h2_ref[...] = h2
    hm2_ref[...] = _dg00(wmsg_ref[0:D, :], h2)


def _tc_update(hT, aggT, W_upd, W_msg):
    return pl.pallas_call(
        _tc_update_body,
        out_shape=[
            jax.ShapeDtypeStruct((D, N), F32),
            jax.ShapeDtypeStruct((D, N), F32),
        ],
    )(hT, aggT, W_upd, W_msg)


def _tc_readout_body(h_ref, gid_ref, nt_ref, wl_ref, wd_ref, wo_ref,
                     lat_ref, nll_ref):
    gid = gid_ref[...]
    growz = lax.broadcasted_iota(jnp.int32, (G, N), 0)
    ohG = (growz == gid).astype(F32)
    hT = h_ref[...]
    graph_sumT = _dg11(hT, ohG)
    ones = jnp.ones((1, N), F32)
    counts = _dg11(ones, ohG)
    graph_repT = graph_sumT / jnp.maximum(counts, 1.0)
    latents = _dg00(graph_repT, wl_ref[...])
    lat_ref[...] = latents
    decT = jnp.maximum(
        lax.dot_general(wd_ref[...], latents, (((0,), (1,)), ((), ())),
                        preferred_element_type=F32), 0.0)
    logitsGT = _dg00(wo_ref[...], decT)
    m = jnp.max(logitsGT, axis=0, keepdims=True)
    s = jnp.sum(jnp.exp(logitsGT - m), axis=0, keepdims=True)
    lse = m + jnp.log(s)
    nt = nt_ref[...]
    trowz = lax.broadcasted_iota(jnp.int32, (NV, N), 0)
    ohT = (trowz == nt).astype(F32)
    J = _dg11(ohT, ohG)
    term1 = jnp.sum(counts * lse)
    term2 = jnp.sum(J * logitsGT)
    nll_ref[...] = ((term1 - term2) / jnp.float32(N)).reshape(1, 1)


def _tc_readout(hT, gid2, nt2, W_latent, W_dec, W_out):
    return pl.pallas_call(
        _tc_readout_body,
        out_shape=[
            jax.ShapeDtypeStruct((G, LATENT), F32),
            jax.ShapeDtypeStruct((1, 1), F32),
        ],
    )(hT, gid2, nt2, W_latent, W_dec, W_out)


ROWS_PER_TEC = 4
CHUNK = 6400
NCHUNK = E // CHUNK
LANES = 16

_sc_mesh = plsc.VectorSubcoreMesh(core_axis_name="c", subcore_axis_name="s")


def _sc_edge_body(hm_hbm, eb_hbm, src_hbm, dst_hbm, typ_hbm, agg_hbm,
                  hms, ags, ebs, bufs, sems):
    wid = lax.axis_index("s") * 2 + lax.axis_index("c")
    row = wid * ROWS_PER_TEC

    for c in range(ROWS_PER_TEC):
        pltpu.sync_copy(hm_hbm.at[row + c], hms[c])
        pltpu.sync_copy(eb_hbm.at[row + c], ebs[c])

    zv = jnp.zeros((LANES,), F32)

    def _zero(j, _):
        o = j * LANES
        for c in range(ROWS_PER_TEC):
            ags[c][pl.ds(o, LANES)] = zv
        return 0

    lax.fori_loop(0, N // LANES, _zero, 0)

    def _issue(i, slot):
        base = i * CHUNK
        sb, db, tb = bufs[slot]
        ss, sd, st = sems[slot]
        pltpu.async_copy(src_hbm.at[pl.ds(base, CHUNK)], sb, ss)
        pltpu.async_copy(dst_hbm.at[pl.ds(base, CHUNK)], db, sd)
        pltpu.async_copy(typ_hbm.at[pl.ds(base, CHUNK)], tb, st)

    def _wait(i, slot):
        base = i * CHUNK
        sb, db, tb = bufs[slot]
        ss, sd, st = sems[slot]
        pltpu.make_async_copy(src_hbm.at[pl.ds(base, CHUNK)], sb, ss).wait()
        pltpu.make_async_copy(dst_hbm.at[pl.ds(base, CHUNK)], db, sd).wait()
        pltpu.make_async_copy(typ_hbm.at[pl.ds(base, CHUNK)], tb, st).wait()

    def _compute(slot):
        sb, db, tb = bufs[slot]

        def _vec(j, _):
            o = j * LANES
            s16 = sb[pl.ds(o, LANES)]
            d16 = db[pl.ds(o, LANES)]
            t16 = tb[pl.ds(o, LANES)]
            for c in range(ROWS_PER_TEC):
                hv = plsc.load_gather(hms[c], [s16])
                ev = plsc.load_gather(ebs[c], [t16])
                msg = jnp.maximum(hv + ev, 0.0)
                plsc.addupdate_scatter(ags[c], [d16], msg)
            return 0

        lax.fori_loop(0, CHUNK // LANES, _vec, 0, unroll=8)

    _issue(0, 0)
    _issue(1, 1)

    def _pair(k, _):
        i0 = 2 * k
        _wait(i0, 0)
        _compute(0)

        @pl.when(i0 + 2 < NCHUNK)
        def _():
            _issue(i0 + 2, 0)

        _wait(i0 + 1, 1)
        _compute(1)

        @pl.when(i0 + 3 < NCHUNK)
        def _():
            _issue(i0 + 3, 1)

        return 0

    lax.fori_loop(0, NCHUNK // 2, _pair, 0)

    for c in range(ROWS_PER_TEC):
        pltpu.sync_copy(ags[c], agg_hbm.at[row + c])


@functools.partial(
    pl.kernel,
    out_type=jax.ShapeDtypeStruct((D, N), F32),
    mesh=_sc_mesh,
    compiler_params=pltpu.CompilerParams(needs_layout_passes=False),
    scratch_types=[
        [pltpu.VMEM((N,), F32)] * ROWS_PER_TEC,
        [pltpu.VMEM((N,), F32)] * ROWS_PER_TEC,
        [pltpu.VMEM((EVP,), F32)] * ROWS_PER_TEC,
        [[pltpu.VMEM((CHUNK,), jnp.int32)] * 3] * 2,
        [[pltpu.SemaphoreType.DMA] * 3] * 2,
    ],
)
def _sc_edge_pass(hm_hbm, eb_hbm, src_hbm, dst_hbm, typ_hbm, agg_hbm,
                  hms, ags, ebs, bufs, sems):
    _sc_edge_body(hm_hbm, eb_hbm, src_hbm, dst_hbm, typ_hbm, agg_hbm,
                  hms, ags, ebs, bufs, sems)


def kernel(node_types, edge_index, edge_types, graph_ids,
           node_embed, edge_embed, W_msg, W_upd, W_latent, W_dec, W_out):
    src = edge_index[0].astype(jnp.int32)
    dst = edge_index[1].astype(jnp.int32)
    typ = edge_types.astype(jnp.int32)
    nt2 = node_types.astype(jnp.int32).reshape(1, N)
    gid2 = graph_ids.astype(jnp.int32).reshape(1, N)
    edge_embed_pad = jnp.pad(edge_embed, ((0, EVP - EV), (0, 0)))

    hT, hmT, ebT = _tc_init(nt2, node_embed, edge_embed_pad, W_msg)
    for _ in range(HOPS):
        aggT = _sc_edge_pass(hmT, ebT, src, dst, typ)
        hT, hmT = _tc_update(hT, aggT, W_upd, W_msg)
    latents, nll = _tc_readout(hT, gid2, nt2, W_latent, W_dec, W_out)
    return latents, nll.reshape(()), jnp.zeros((), F32)


# speedup vs baseline: 2.4109x; 2.4109x over previous
import functools

import jax
import jax.numpy as jnp
from jax import lax
from jax.experimental import pallas as pl
from jax.experimental.pallas import tpu as pltpu
from jax.experimental.pallas import tpu_sc as plsc

N = 10000
E = 320000
G = 256
NV = 100
EV = 20
EVP = 32
D = 128
LATENT = 64
HOPS = 3

F32 = jnp.float32


def _dg00(a, b):
    return lax.dot_general(a, b, (((0,), (0,)), ((), ())),
                           preferred_element_type=F32)


def _dg11(a, b):
    return lax.dot_general(a, b, (((1,), (1,)), ((), ())),
                           preferred_element_type=F32)


def _tc_init_body(nt_ref, nemb_ref, eemb_ref, wmsg_ref,
                  hT_ref, hmT_ref, ebT_ref):
    nt = nt_ref[...]
    rows = lax.broadcasted_iota(jnp.int32, (NV, N), 0)
    ohT = (rows == nt).astype(F32)
    hT = _dg00(nemb_ref[...], ohT)
    wm1 = wmsg_ref[0:D, :]
    wm2 = wmsg_ref[D:2 * D, :]
    hT_ref[...] = hT
    hmT_ref[...] = _dg00(wm1, hT)
    ebT_ref[...] = lax.dot_general(wm2, eemb_ref[...],
                                   (((0,), (1,)), ((), ())),
                                   preferred_element_type=F32)


def _tc_init(nt2, node_embed, edge_embed_pad, W_msg):
    return pl.pallas_call(
        _tc_init_body,
        out_shape=[
            jax.ShapeDtypeStruct((D, N), F32),
            jax.ShapeDtypeStruct((D, N), F32),
            jax.ShapeDtypeStruct((D, EVP), F32),
        ],
    )(nt2, node_embed, edge_embed_pad, W_msg)


def _tc_update_body(h_ref, agg_ref, wupd_ref, wmsg_ref, h2_ref, hm2_ref):
    wu1 = wupd_ref[0:D, :]
    wu2 = wupd_ref[D:2 * D, :]
    h2 = jnp.maximum(_dg00(wu1, h_ref[...]) + _dg00(wu2, agg_ref[...]), 0.0)
    h2_ref[...] = h2
    hm2_ref[...] = _dg00(wmsg_ref[0:D, :], h2)


def _tc_update(hT, aggT, W_upd, W_msg):
    return pl.pallas_call(
        _tc_update_body,
        out_shape=[
            jax.ShapeDtypeStruct((D, N), F32),
            jax.ShapeDtypeStruct((D, N), F32),
        ],
    )(hT, aggT, W_upd, W_msg)


def _tc_readout_body(h_ref, gid_ref, nt_ref, wl_ref, wd_ref, wo_ref,
                     lat_ref, nll_ref):
    gid = gid_ref[...]
    growz = lax.broadcasted_iota(jnp.int32, (G, N), 0)
    ohG = (growz == gid).astype(F32)
    hT = h_ref[...]
    graph_sumT = _dg11(hT, ohG)
    ones = jnp.ones((1, N), F32)
    counts = _dg11(ones, ohG)
    graph_repT = graph_sumT / jnp.maximum(counts, 1.0)
    latents = _dg00(graph_repT, wl_ref[...])
    lat_ref[...] = latents
    decT = jnp.maximum(
        lax.dot_general(wd_ref[...], latents, (((0,), (1,)), ((), ())),
                        preferred_element_type=F32), 0.0)
    logitsGT = _dg00(wo_ref[...], decT)
    m = jnp.max(logitsGT, axis=0, keepdims=True)
    s = jnp.sum(jnp.exp(logitsGT - m), axis=0, keepdims=True)
    lse = m + jnp.log(s)
    nt = nt_ref[...]
    trowz = lax.broadcasted_iota(jnp.int32, (NV, N), 0)
    ohT = (trowz == nt).astype(F32)
    J = _dg11(ohT, ohG)
    term1 = jnp.sum(counts * lse)
    term2 = jnp.sum(J * logitsGT)
    nll_ref[...] = ((term1 - term2) / jnp.float32(N)).reshape(1, 1)


def _tc_readout(hT, gid2, nt2, W_latent, W_dec, W_out):
    return pl.pallas_call(
        _tc_readout_body,
        out_shape=[
            jax.ShapeDtypeStruct((G, LATENT), F32),
            jax.ShapeDtypeStruct((1, 1), F32),
        ],
    )(hT, gid2, nt2, W_latent, W_dec, W_out)


ROWS_PER_TEC = 4
CHUNK = 6400
NCHUNK = E // CHUNK
LANES = 16

_sc_mesh = plsc.VectorSubcoreMesh(core_axis_name="c", subcore_axis_name="s")


def _sc_edge_body(hm_hbm, eb_hbm, src_hbm, dst_hbm, typ_hbm, agg_hbm,
                  hms, ags, ebs, bufs, sems):
    wid = lax.axis_index("s") * 2 + lax.axis_index("c")
    row = wid * ROWS_PER_TEC

    for c in range(ROWS_PER_TEC):
        pltpu.sync_copy(hm_hbm.at[row + c], hms[c])
        pltpu.sync_copy(eb_hbm.at[row + c], ebs[c])

    zv = jnp.zeros((LANES,), F32)

    def _zero(j, _):
        o = j * LANES
        for c in range(ROWS_PER_TEC):
            ags[c][pl.ds(o, LANES)] = zv
        return 0

    lax.fori_loop(0, N // LANES, _zero, 0)

    def _issue(i, slot):
        base = i * CHUNK
        sb, db, tb = bufs[slot]
        ss, sd, st = sems[slot]
        pltpu.async_copy(src_hbm.at[pl.ds(base, CHUNK)], sb, ss)
        pltpu.async_copy(dst_hbm.at[pl.ds(base, CHUNK)], db, sd)
        pltpu.async_copy(typ_hbm.at[pl.ds(base, CHUNK)], tb, st)

    def _wait(i, slot):
        base = i * CHUNK
        sb, db, tb = bufs[slot]
        ss, sd, st = sems[slot]
        pltpu.make_async_copy(src_hbm.at[pl.ds(base, CHUNK)], sb, ss).wait()
        pltpu.make_async_copy(dst_hbm.at[pl.ds(base, CHUNK)], db, sd).wait()
        pltpu.make_async_copy(typ_hbm.at[pl.ds(base, CHUNK)], tb, st).wait()

    def _compute(slot):
        sb, db, tb = bufs[slot]

        @plsc.parallel_loop(0, CHUNK // LANES, unroll=8)
        def _vec(j):
            o = j * LANES
            s16 = sb[pl.ds(o, LANES)]
            d16 = db[pl.ds(o, LANES)]
            t16 = tb[pl.ds(o, LANES)]
            for c in range(ROWS_PER_TEC):
                hv = plsc.load_gather(hms[c], [s16])
                ev = plsc.load_gather(ebs[c], [t16])
                msg = jnp.maximum(hv + ev, 0.0)
                plsc.addupdate_scatter(ags[c], [d16], msg)

    _issue(0, 0)
    _issue(1, 1)

    def _pair(k, _):
        i0 = 2 * k
        _wait(i0, 0)
        _compute(0)

        @pl.when(i0 + 2 < NCHUNK)
        def _():
            _issue(i0 + 2, 0)

        _wait(i0 + 1, 1)
        _compute(1)

        @pl.when(i0 + 3 < NCHUNK)
        def _():
            _issue(i0 + 3, 1)

        return 0

    lax.fori_loop(0, NCHUNK // 2, _pair, 0)

    for c in range(ROWS_PER_TEC):
        pltpu.sync_copy(ags[c], agg_hbm.at[row + c])


@functools.partial(
    pl.kernel,
    out_type=jax.ShapeDtypeStruct((D, N), F32),
    mesh=_sc_mesh,
    compiler_params=pltpu.CompilerParams(needs_layout_passes=False),
    scratch_types=[
        [pltpu.VMEM((N,), F32)] * ROWS_PER_TEC,
        [pltpu.VMEM((N,), F32)] * ROWS_PER_TEC,
        [pltpu.VMEM((EVP,), F32)] * ROWS_PER_TEC,
        [[pltpu.VMEM((CHUNK,), jnp.int32)] * 3] * 2,
        [[pltpu.SemaphoreType.DMA] * 3] * 2,
    ],
)
def _sc_edge_pass(hm_hbm, eb_hbm, src_hbm, dst_hbm, typ_hbm, agg_hbm,
                  hms, ags, ebs, bufs, sems):
    _sc_edge_body(hm_hbm, eb_hbm, src_hbm, dst_hbm, typ_hbm, agg_hbm,
                  hms, ags, ebs, bufs, sems)


def kernel(node_types, edge_index, edge_types, graph_ids,
           node_embed, edge_embed, W_msg, W_upd, W_latent, W_dec, W_out):
    src = edge_index[0].astype(jnp.int32)
    dst = edge_index[1].astype(jnp.int32)
    typ = edge_types.astype(jnp.int32)
    nt2 = node_types.astype(jnp.int32).reshape(1, N)
    gid2 = graph_ids.astype(jnp.int32).reshape(1, N)
    edge_embed_pad = jnp.pad(edge_embed, ((0, EVP - EV), (0, 0)))

    hT, hmT, ebT = _tc_init(nt2, node_embed, edge_embed_pad, W_msg)
    for _ in range(HOPS):
        aggT = _sc_edge_pass(hmT, ebT, src, dst, typ)
        hT, hmT = _tc_update(hT, aggT, W_upd, W_msg)
    latents, nll = _tc_readout(hT, gid2, nt2, W_latent, W_dec, W_out)
    return latents, nll.reshape(()), jnp.zeros((), F32)


# speedup vs baseline: 3.1464x; 1.3051x over previous
import functools

import jax
import jax.numpy as jnp
from jax import lax
from jax.experimental import pallas as pl
from jax.experimental.pallas import tpu as pltpu
from jax.experimental.pallas import tpu_sc as plsc

N = 10000
E = 320000
G = 256
NV = 100
EV = 20
EVP = 32
D = 128
DH = D // 2
LATENT = 64
HOPS = 3

F32 = jnp.float32


def _dg00(a, b):
    return lax.dot_general(a, b, (((0,), (0,)), ((), ())),
                           preferred_element_type=F32)


def _dg11(a, b):
    return lax.dot_general(a, b, (((1,), (1,)), ((), ())),
                           preferred_element_type=F32)


def _pack_pairs(x):
    lo = lax.bitcast_convert_type(x[0:DH].astype(jnp.bfloat16),
                                  jnp.uint16).astype(jnp.uint32)
    hi = lax.bitcast_convert_type(x[DH:D].astype(jnp.bfloat16),
                                  jnp.uint16).astype(jnp.uint32)
    return lax.bitcast_convert_type(lo | (hi << jnp.uint32(16)), jnp.int32)


def _tc_init_body(nt_ref, nemb_ref, eemb_ref, wmsg_ref,
                  hT_ref, hmp_ref, ebp_ref):
    nt = nt_ref[...]
    rows = lax.broadcasted_iota(jnp.int32, (NV, N), 0)
    ohT = (rows == nt).astype(F32)
    hT = _dg00(nemb_ref[...], ohT)
    wm1 = wmsg_ref[0:D, :]
    wm2 = wmsg_ref[D:2 * D, :]
    hT_ref[...] = hT
    hmp_ref[...] = _pack_pairs(_dg00(wm1, hT))
    ebT = lax.dot_general(wm2, eemb_ref[...], (((0,), (1,)), ((), ())),
                          preferred_element_type=F32)
    ebp_ref[...] = _pack_pairs(ebT)


def _tc_init(nt2, node_embed, edge_embed_pad, W_msg):
    return pl.pallas_call(
        _tc_init_body,
        out_shape=[
            jax.ShapeDtypeStruct((D, N), F32),
            jax.ShapeDtypeStruct((DH, N), jnp.int32),
            jax.ShapeDtypeStruct((DH, EVP), jnp.int32),
        ],
    )(nt2, node_embed, edge_embed_pad, W_msg)


def _tc_update_body(h_ref, agg_ref, wupd_ref, wmsg_ref, h2_ref, hmp2_ref):
    wu1 = wupd_ref[0:D, :]
    wu2 = wupd_ref[D:2 * D, :]
    h2 = jnp.maximum(_dg00(wu1, h_ref[...]) + _dg00(wu2, agg_ref[...]), 0.0)
    h2_ref[...] = h2
    hmp2_ref[...] = _pack_pairs(_dg00(wmsg_ref[0:D, :], h2))


def _tc_update(hT, aggT, W_upd, W_msg):
    return pl.pallas_call(
        _tc_update_body,
        out_shape=[
            jax.ShapeDtypeStruct((D, N), F32),
            jax.ShapeDtypeStruct((DH, N), jnp.int32),
        ],
    )(hT, aggT, W_upd, W_msg)


def _tc_readout_body(h_ref, gid_ref, nt_ref, wl_ref, wd_ref, wo_ref,
                     lat_ref, nll_ref):
    gid = gid_ref[...]
    growz = lax.broadcasted_iota(jnp.int32, (G, N), 0)
    ohG = (growz == gid).astype(F32)
    hT = h_ref[...]
    graph_sumT = _dg11(hT, ohG)
    ones = jnp.ones((1, N), F32)
    counts = _dg11(ones, ohG)
    graph_repT = graph_sumT / jnp.maximum(counts, 1.0)
    latents = _dg00(graph_repT, wl_ref[...])
    lat_ref[...] = latents
    decT = jnp.maximum(
        lax.dot_general(wd_ref[...], latents, (((0,), (1,)), ((), ())),
                        preferred_element_type=F32), 0.0)
    logitsGT = _dg00(wo_ref[...], decT)
    m = jnp.max(logitsGT, axis=0, keepdims=True)
    s = jnp.sum(jnp.exp(logitsGT - m), axis=0, keepdims=True)
    lse = m + jnp.log(s)
    nt = nt_ref[...]
    trowz = lax.broadcasted_iota(jnp.int32, (NV, N), 0)
    ohT = (trowz == nt).astype(F32)
    J = _dg11(ohT, ohG)
    term1 = jnp.sum(counts * lse)
    term2 = jnp.sum(J * logitsGT)
    nll_ref[...] = ((term1 - term2) / jnp.float32(N)).reshape(1, 1)


def _tc_readout(hT, gid2, nt2, W_latent, W_dec, W_out):
    return pl.pallas_call(
        _tc_readout_body,
        out_shape=[
            jax.ShapeDtypeStruct((G, LATENT), F32),
            jax.ShapeDtypeStruct((1, 1), F32),
        ],
    )(hT, gid2, nt2, W_latent, W_dec, W_out)


PAIRS_PER_TEC = 2
CHUNK = 6400
NCHUNK = E // CHUNK
LANES = 16
SRC_MASK = (1 << 14) - 1

_sc_mesh = plsc.VectorSubcoreMesh(core_axis_name="c", subcore_axis_name="s")


def _sc_edge_body(hm_hbm, eb_hbm, pst_hbm, dst_hbm, agg_hbm,
                  hmp, ags, ebp, bufs, sems):
    wid = lax.axis_index("s") * 2 + lax.axis_index("c")
    row = wid * PAIRS_PER_TEC

    for q in range(PAIRS_PER_TEC):
        pltpu.sync_copy(hm_hbm.at[row + q], hmp[q])
        pltpu.sync_copy(eb_hbm.at[row + q], ebp[q])

    zv = jnp.zeros((LANES,), F32)

    @plsc.parallel_loop(0, N // LANES)
    def _zero(j):
        o = j * LANES
        for c in range(2 * PAIRS_PER_TEC):
            ags[c][pl.ds(o, LANES)] = zv

    def _issue(i, slot):
        base = i * CHUNK
        pb, db = bufs[slot]
        sp, sd = sems[slot]
        pltpu.async_copy(pst_hbm.at[pl.ds(base, CHUNK)], pb, sp)
        pltpu.async_copy(dst_hbm.at[pl.ds(base, CHUNK)], db, sd)

    def _wait(i, slot):
        base = i * CHUNK
        pb, db = bufs[slot]
        sp, sd = sems[slot]
        pltpu.make_async_copy(pst_hbm.at[pl.ds(base, CHUNK)], pb, sp).wait()
        pltpu.make_async_copy(dst_hbm.at[pl.ds(base, CHUNK)], db, sd).wait()

    def _compute(slot):
        pb, db = bufs[slot]

        @plsc.parallel_loop(0, CHUNK // LANES, unroll=8)
        def _vec(j):
            o = j * LANES
            p16 = pb[pl.ds(o, LANES)]
            d16 = db[pl.ds(o, LANES)]
            s16 = jnp.bitwise_and(p16, SRC_MASK)
            t16 = jnp.right_shift(p16, 14)
            for q in range(PAIRS_PER_TEC):
                hw = plsc.load_gather(hmp[q], [s16])
                hlo, hhi = plsc.unpack(plsc.bitcast(hw, jnp.bfloat16),
                                       format=plsc.PackFormat.INTERLEAVED)
                ew = plsc.load_gather(ebp[q], [t16])
                elo, ehi = plsc.unpack(plsc.bitcast(ew, jnp.bfloat16),
                                       format=plsc.PackFormat.INTERLEAVED)
                plsc.addupdate_scatter(ags[q], [d16],
                                       jnp.maximum(hlo + elo, 0.0))
                plsc.addupdate_scatter(ags[PAIRS_PER_TEC + q], [d16],
                                       jnp.maximum(hhi + ehi, 0.0))

    _issue(0, 0)
    _issue(1, 1)

    def _pair(k, _):
        i0 = 2 * k
        _wait(i0, 0)
        _compute(0)

        @pl.when(i0 + 2 < NCHUNK)
        def _():
            _issue(i0 + 2, 0)

        _wait(i0 + 1, 1)
        _compute(1)

        @pl.when(i0 + 3 < NCHUNK)
        def _():
            _issue(i0 + 3, 1)

        return 0

    lax.fori_loop(0, NCHUNK // 2, _pair, 0)

    for q in range(PAIRS_PER_TEC):
        pltpu.sync_copy(ags[q], agg_hbm.at[row + q])
        pltpu.sync_copy(ags[PAIRS_PER_TEC + q], agg_hbm.at[DH + row + q])


@functools.partial(
    pl.kernel,
    out_type=jax.ShapeDtypeStruct((D, N), F32),
    mesh=_sc_mesh,
    compiler_params=pltpu.CompilerParams(needs_layout_passes=False),
    scratch_types=[
        [pltpu.VMEM((N,), jnp.int32)] * PAIRS_PER_TEC,
        [pltpu.VMEM((N,), F32)] * (2 * PAIRS_PER_TEC),
        [pltpu.VMEM((EVP,), jnp.int32)] * PAIRS_PER_TEC,
        [[pltpu.VMEM((CHUNK,), jnp.int32)] * 2] * 2,
        [[pltpu.SemaphoreType.DMA] * 2] * 2,
    ],
)
def _sc_edge_pass(hm_hbm, eb_hbm, pst_hbm, dst_hbm, agg_hbm,
                  hmp, ags, ebp, bufs, sems):
    _sc_edge_body(hm_hbm, eb_hbm, pst_hbm, dst_hbm, agg_hbm,
                  hmp, ags, ebp, bufs, sems)


def kernel(node_types, edge_index, edge_types, graph_ids,
           node_embed, edge_embed, W_msg, W_upd, W_latent, W_dec, W_out):
    src = edge_index[0].astype(jnp.int32)
    dst = edge_index[1].astype(jnp.int32)
    typ = edge_types.astype(jnp.int32)
    pst = jnp.bitwise_or(src, jnp.left_shift(typ, 14))
    nt2 = node_types.astype(jnp.int32).reshape(1, N)
    gid2 = graph_ids.astype(jnp.int32).reshape(1, N)
    edge_embed_pad = jnp.pad(edge_embed, ((0, EVP - EV), (0, 0)))

    hT, hmp, ebp = _tc_init(nt2, node_embed, edge_embed_pad, W_msg)
    for _ in range(HOPS):
        aggT = _sc_edge_pass(hmp, ebp, pst, dst)
        hT, hmp = _tc_update(hT, aggT, W_upd, W_msg)
    latents, nll = _tc_readout(hT, gid2, nt2, W_latent, W_dec, W_out)
    return latents, nll.reshape(()), jnp.zeros((), F32)
